# baseline (speedup 1.0000x reference)
import functools

import jax
import jax.numpy as jnp
from jax import lax
from jax.experimental import pallas as pl
from jax.experimental.pallas import tpu as pltpu
from jax.experimental.pallas import tpu_sc as plsc

N, E, D = 10000, 320000, 128

NC, NS = 2, 16
NW = NC * NS
EPW = E // NW
CH = 40
NCH = EPW // CH
RING = 4
PF = 2
EPT = E // NS
NCHT = EPT // CH
SCH = 80
SNCH = EPW // SCH
SRING = 4
SPF = 2
RPT = N // NS


@functools.lru_cache(maxsize=None)
def _build_sc_kernels():
    mesh = plsc.VectorSubcoreMesh(core_axis_name="c", subcore_axis_name="s")

    @functools.partial(
        pl.kernel,
        mesh=mesh,
        out_type=(jax.ShapeDtypeStruct((E, D), jnp.float32),
                  jax.ShapeDtypeStruct((E, D), jnp.float32)),
        scratch_types=[
            pltpu.VMEM((EPT,), jnp.int32),
            pltpu.VMEM((RING, CH, D), jnp.float32),
            pltpu.VMEM_SHARED((N, D), jnp.float32),
            pltpu.SemaphoreType.DMA,
            pltpu.SemaphoreType.DMA,
        ],
    )
    def _sc_gather(pr4_hbm, ps4_hbm, recv2_hbm, send2_hbm, gr_hbm, gs_hbm,
                   idx_v, rbuf, table_sp, sem_g, sem_w):
        cid = lax.axis_index("c")
        sid = lax.axis_index("s")
        ebase = sid * EPT

        @pl.when(cid == 0)
        def _():
            pltpu.sync_copy(pr4_hbm.at[sid], table_sp.at[pl.ds(sid * RPT, RPT)])
            pltpu.sync_copy(recv2_hbm.at[sid], idx_v)

        @pl.when(cid == 1)
        def _():
            pltpu.sync_copy(ps4_hbm.at[sid], table_sp.at[pl.ds(sid * RPT, RPT)])
            pltpu.sync_copy(send2_hbm.at[sid], idx_v)

        plsc.subcore_barrier()

        def run(g_hbm):
            def issue_gather(c):
                sl = lax.rem(c, RING)
                pltpu.async_copy(table_sp.at[idx_v.at[pl.ds(c * CH, CH)]],
                                 rbuf.at[sl], sem_g)

            for c0 in range(PF):
                issue_gather(c0)

            def chunk(c, carry):
                sl = lax.rem(c, RING)
                pltpu.make_async_copy(table_sp.at[idx_v.at[pl.ds(0, CH)]],
                                      rbuf.at[sl], sem_g).wait()
                pltpu.async_copy(rbuf.at[sl],
                                 g_hbm.at[pl.ds(ebase + c * CH, CH)], sem_w)

                @pl.when(c + PF >= RING)
                def _():
                    off = ebase + (c + PF - RING) * CH
                    slp = lax.rem(c + PF, RING)
                    pltpu.make_async_copy(rbuf.at[slp],
                                          g_hbm.at[pl.ds(off, CH)],
                                          sem_w).wait()

                @pl.when(c + PF < NCHT)
                def _():
                    issue_gather(c + PF)

                return carry

            lax.fori_loop(0, NCHT, chunk, 0)

            for t in range(RING - PF):
                c = NCHT - (RING - PF) + t
                sl = c % RING
                pltpu.make_async_copy(rbuf.at[sl],
                                      g_hbm.at[pl.ds(ebase + c * CH, CH)],
                                      sem_w).wait()

        @pl.when(cid == 0)
        def _():
            run(gr_hbm)

        @pl.when(cid == 1)
        def _():
            run(gs_hbm)

    @functools.partial(
        pl.kernel,
        mesh=mesh,
        out_type=jax.ShapeDtypeStruct((NC, NS, RPT, D), jnp.float32),
        scratch_types=[
            pltpu.VMEM((SRING, SCH), jnp.int32),
            pltpu.VMEM((SRING, SCH, D), jnp.float32),
            pltpu.VMEM_SHARED((N, D), jnp.float32),
            pltpu.SemaphoreType.DMA,
            pltpu.SemaphoreType.DMA,
            pltpu.SemaphoreType.DMA,
        ],
    )
    def _sc_scatter(newe_hbm, recv3_hbm, zeros_hbm, out_hbm,
                    idx_v, rows, inbox_sh, sem_i, sem_l, sem_sc):
        cid = lax.axis_index("c")
        sid = lax.axis_index("s")
        wid = sid * NC + cid
        ebase = wid * EPW

        pltpu.sync_copy(zeros_hbm, inbox_sh.at[pl.ds(sid * RPT, RPT)])
        plsc.subcore_barrier()

        def issue_loads(c):
            sl = lax.rem(c, SRING)
            pltpu.async_copy(recv3_hbm.at[wid, c], idx_v.at[sl], sem_i)
            pltpu.async_copy(newe_hbm.at[pl.ds(ebase + c * SCH, SCH)],
                             rows.at[sl], sem_l)

        for c0 in range(SPF):
            issue_loads(c0)

        def chunk(c, carry):
            sl = lax.rem(c, SRING)
            pltpu.make_async_copy(recv3_hbm.at[wid, c], idx_v.at[sl],
                                  sem_i).wait()
            pltpu.make_async_copy(newe_hbm.at[pl.ds(ebase + c * SCH, SCH)],
                                  rows.at[sl], sem_l).wait()
            pltpu.async_copy(rows.at[sl], inbox_sh.at[idx_v.at[sl]], sem_sc,
                             add=True)

            @pl.when(c + SPF >= SRING)
            def _():
                slp = lax.rem(c + SPF, SRING)
                pltpu.make_async_copy(rows.at[slp],
                                      inbox_sh.at[idx_v.at[slp]],
                                      sem_sc).wait()

            @pl.when(c + SPF < SNCH)
            def _():
                issue_loads(c + SPF)

            return carry

        lax.fori_loop(0, SNCH, chunk, 0)

        for t in range(SRING - SPF):
            c = SNCH - (SRING - SPF) + t
            sl = c % SRING
            pltpu.make_async_copy(rows.at[sl], inbox_sh.at[idx_v.at[sl]],
                                  sem_sc).wait()

        plsc.subcore_barrier()
        pltpu.sync_copy(inbox_sh.at[pl.ds(sid * RPT, RPT)],
                        out_hbm.at[cid, sid])

    return _sc_gather, _sc_scatter


def _edge_body(e_ref, gr_ref, gs_ref, w1e_ref, w1r_ref, w1s_ref, b1_ref,
               w2_ref, b2_ref, out_ref):
    acc = (jnp.dot(e_ref[...], w1e_ref[...], preferred_element_type=jnp.float32)
           + jnp.dot(gr_ref[...], w1r_ref[...], preferred_element_type=jnp.float32)
           + jnp.dot(gs_ref[...], w1s_ref[...], preferred_element_type=jnp.float32))
    h = jnp.maximum(acc + b1_ref[...], 0.0)
    out_ref[...] = jnp.dot(h, w2_ref[...], preferred_element_type=jnp.float32) + b2_ref[...]


def _node_body(x_ref, i0_ref, i1_ref, wa_ref, wb_ref, b1_ref, w2_ref, b2_ref, out_ref):
    x = x_ref[...]
    inbox = i0_ref[...] + i1_ref[...]
    h = jnp.maximum(
        jnp.dot(x, wa_ref[...], preferred_element_type=jnp.float32)
        + jnp.dot(inbox, wb_ref[...], preferred_element_type=jnp.float32)
        + b1_ref[...], 0.0)
    out_ref[...] = x + jnp.dot(h, w2_ref[...], preferred_element_type=jnp.float32) + b2_ref[...]


BE = 2560


def kernel(nodes, edges, senders, receivers, edge_W1, edge_b1, edge_W2, edge_b2,
           node_W1, node_b1, node_W2, node_b2):
    x = nodes.reshape(N, D)
    e = edges.reshape(E, D)
    recv = receivers.astype(jnp.int32)
    send = senders.astype(jnp.int32)
    recv2 = recv.reshape(NS, EPT)
    send2 = send.reshape(NS, EPT)
    recv3 = recv.reshape(NW, SNCH, SCH)

    w1e = edge_W1[:D]
    w1r = edge_W1[D:2 * D]
    w1s = edge_W1[2 * D:]
    wn1a = node_W1[:D]
    wn1b = node_W1[D:]

    sc_gather, sc_scatter = _build_sc_kernels()
    x4 = x.reshape(NS, RPT, D)
    g_r, g_s = sc_gather(x4, x4, recv2, send2)

    b1r = edge_b1.reshape(1, D)
    b2r = edge_b2.reshape(1, D)
    new_edges = pl.pallas_call(
        _edge_body,
        grid=(E // BE,),
        in_specs=[
            pl.BlockSpec((BE, D), lambda i: (i, 0)),
            pl.BlockSpec((BE, D), lambda i: (i, 0)),
            pl.BlockSpec((BE, D), lambda i: (i, 0)),
            pl.BlockSpec((D, D), lambda i: (0, 0)),
            pl.BlockSpec((D, D), lambda i: (0, 0)),
            pl.BlockSpec((D, D), lambda i: (0, 0)),
            pl.BlockSpec((1, D), lambda i: (0, 0)),
            pl.BlockSpec((D, D), lambda i: (0, 0)),
            pl.BlockSpec((1, D), lambda i: (0, 0)),
        ],
        out_specs=pl.BlockSpec((BE, D), lambda i: (i, 0)),
        out_shape=jax.ShapeDtypeStruct((E, D), jnp.float32),
    )(e, g_r, g_s, w1e, w1r, w1s, b1r, edge_W2, b2r)

    zrows = jnp.zeros((RPT, D), jnp.float32)
    partials = sc_scatter(new_edges, recv3, zrows).reshape(NC, N, D)

    bn1r = node_b1.reshape(1, D)
    bn2r = node_b2.reshape(1, D)
    BN = 2000
    new_nodes = pl.pallas_call(
        _node_body,
        grid=(N // BN,),
        in_specs=[
            pl.BlockSpec((BN, D), lambda i: (i, 0)),
            pl.BlockSpec((BN, D), lambda i: (i, 0)),
            pl.BlockSpec((BN, D), lambda i: (i, 0)),
            pl.BlockSpec((D, D), lambda i: (0, 0)),
            pl.BlockSpec((D, D), lambda i: (0, 0)),
            pl.BlockSpec((1, D), lambda i: (0, 0)),
            pl.BlockSpec((D, D), lambda i: (0, 0)),
            pl.BlockSpec((1, D), lambda i: (0, 0)),
        ],
        out_specs=pl.BlockSpec((BN, D), lambda i: (i, 0)),
        out_shape=jax.ShapeDtypeStruct((N, D), jnp.float32),
    )(x, partials[0], partials[1], wn1a, wn1b, bn1r, node_W2, bn2r)

    return (new_nodes.reshape(1, N, D), new_edges.reshape(1, E, D))


# speedup vs baseline: 1.0199x; 1.0199x over previous
import functools

import jax
import jax.numpy as jnp
from jax import lax
from jax.experimental import pallas as pl
from jax.experimental.pallas import tpu as pltpu
from jax.experimental.pallas import tpu_sc as plsc

N, E, D = 10000, 320000, 128

NC, NS = 2, 16
NW = NC * NS
EPW = E // NW
CH = 40
NCH = EPW // CH
RING = 4
PF = 2
EPT = E // NS
NCHT = EPT // CH
SCH = 80
SNCH = EPW // SCH
SRING = 4
SPF = 2
RPT = N // NS


@functools.lru_cache(maxsize=None)
def _build_sc_kernels():
    mesh = plsc.VectorSubcoreMesh(core_axis_name="c", subcore_axis_name="s")

    @functools.partial(
        pl.kernel,
        mesh=mesh,
        out_type=(jax.ShapeDtypeStruct((E, D), jnp.float32),
                  jax.ShapeDtypeStruct((E, D), jnp.float32)),
        scratch_types=[
            pltpu.VMEM((EPT,), jnp.int32),
            pltpu.VMEM((RING, CH, D), jnp.float32),
            pltpu.VMEM_SHARED((N, D), jnp.float32),
            pltpu.SemaphoreType.DMA,
            pltpu.SemaphoreType.DMA,
        ],
    )
    def _sc_gather(pr4_hbm, ps4_hbm, recv2_hbm, send2_hbm, gr_hbm, gs_hbm,
                   idx_v, rbuf, table_sp, sem_g, sem_w):
        cid = lax.axis_index("c")
        sid = lax.axis_index("s")
        ebase = sid * EPT

        @pl.when(cid == 0)
        def _():
            pltpu.sync_copy(pr4_hbm.at[sid], table_sp.at[pl.ds(sid * RPT, RPT)])
            pltpu.sync_copy(recv2_hbm.at[sid], idx_v)

        @pl.when(cid == 1)
        def _():
            pltpu.sync_copy(ps4_hbm.at[sid], table_sp.at[pl.ds(sid * RPT, RPT)])
            pltpu.sync_copy(send2_hbm.at[sid], idx_v)

        plsc.subcore_barrier()

        def run(g_hbm):
            def issue_gather(c):
                sl = lax.rem(c, RING)
                pltpu.async_copy(table_sp.at[idx_v.at[pl.ds(c * CH, CH)]],
                                 rbuf.at[sl], sem_g)

            for c0 in range(PF):
                issue_gather(c0)

            def chunk(c, carry):
                sl = lax.rem(c, RING)
                pltpu.make_async_copy(table_sp.at[idx_v.at[pl.ds(0, CH)]],
                                      rbuf.at[sl], sem_g).wait()
                pltpu.async_copy(rbuf.at[sl],
                                 g_hbm.at[pl.ds(ebase + c * CH, CH)], sem_w)

                @pl.when(c + PF >= RING)
                def _():
                    off = ebase + (c + PF - RING) * CH
                    slp = lax.rem(c + PF, RING)
                    pltpu.make_async_copy(rbuf.at[slp],
                                          g_hbm.at[pl.ds(off, CH)],
                                          sem_w).wait()

                @pl.when(c + PF < NCHT)
                def _():
                    issue_gather(c + PF)

                return carry

            lax.fori_loop(0, NCHT, chunk, 0)

            for t in range(RING - PF):
                c = NCHT - (RING - PF) + t
                sl = c % RING
                pltpu.make_async_copy(rbuf.at[sl],
                                      g_hbm.at[pl.ds(ebase + c * CH, CH)],
                                      sem_w).wait()

        @pl.when(cid == 0)
        def _():
            run(gr_hbm)

        @pl.when(cid == 1)
        def _():
            run(gs_hbm)

    @functools.partial(
        pl.kernel,
        mesh=mesh,
        out_type=jax.ShapeDtypeStruct((NC, NS, RPT, D), jnp.float32),
        scratch_types=[
            pltpu.VMEM((SRING, SCH), jnp.int32),
            pltpu.VMEM((SRING, SCH, D), jnp.float32),
            pltpu.VMEM_SHARED((N, D), jnp.float32),
            pltpu.SemaphoreType.DMA,
            pltpu.SemaphoreType.DMA,
            pltpu.SemaphoreType.DMA,
        ],
    )
    def _sc_scatter(newe_hbm, recv3_hbm, zeros_hbm, out_hbm,
                    idx_v, rows, inbox_sh, sem_i, sem_l, sem_sc):
        cid = lax.axis_index("c")
        sid = lax.axis_index("s")
        wid = sid * NC + cid
        ebase = wid * EPW

        pltpu.sync_copy(zeros_hbm, inbox_sh.at[pl.ds(sid * RPT, RPT)])
        plsc.subcore_barrier()

        def issue_loads(c):
            sl = lax.rem(c, SRING)
            pltpu.async_copy(recv3_hbm.at[wid, c], idx_v.at[sl], sem_i)
            pltpu.async_copy(newe_hbm.at[pl.ds(ebase + c * SCH, SCH)],
                             rows.at[sl], sem_l)

        for c0 in range(SPF):
            issue_loads(c0)

        def chunk(c, carry):
            sl = lax.rem(c, SRING)
            pltpu.make_async_copy(recv3_hbm.at[wid, c], idx_v.at[sl],
                                  sem_i).wait()
            pltpu.make_async_copy(newe_hbm.at[pl.ds(ebase + c * SCH, SCH)],
                                  rows.at[sl], sem_l).wait()
            pltpu.async_copy(rows.at[sl], inbox_sh.at[idx_v.at[sl]], sem_sc,
                             add=True)

            @pl.when(c + SPF >= SRING)
            def _():
                slp = lax.rem(c + SPF, SRING)
                pltpu.make_async_copy(rows.at[slp],
                                      inbox_sh.at[idx_v.at[slp]],
                                      sem_sc).wait()

            @pl.when(c + SPF < SNCH)
            def _():
                issue_loads(c + SPF)

            return carry

        lax.fori_loop(0, SNCH, chunk, 0)

        for t in range(SRING - SPF):
            c = SNCH - (SRING - SPF) + t
            sl = c % SRING
            pltpu.make_async_copy(rows.at[sl], inbox_sh.at[idx_v.at[sl]],
                                  sem_sc).wait()

        plsc.subcore_barrier()
        pltpu.sync_copy(inbox_sh.at[pl.ds(sid * RPT, RPT)],
                        out_hbm.at[cid, sid])

    return _sc_gather, _sc_scatter


def _prep_body(x_ref, wr_ref, ws_ref, pr_ref, ps_ref):
    x = x_ref[...]
    pr_ref[...] = jnp.dot(x, wr_ref[...], preferred_element_type=jnp.float32)
    ps_ref[...] = jnp.dot(x, ws_ref[...], preferred_element_type=jnp.float32)


def _edge_body(e_ref, gr_ref, gs_ref, w1e_ref, b1_ref, w2_ref, b2_ref, out_ref):
    acc = jnp.dot(e_ref[...], w1e_ref[...], preferred_element_type=jnp.float32)
    h = jnp.maximum(acc + gr_ref[...] + gs_ref[...] + b1_ref[...], 0.0)
    out_ref[...] = jnp.dot(h, w2_ref[...], preferred_element_type=jnp.float32) + b2_ref[...]


def _node_body(x_ref, i0_ref, i1_ref, wa_ref, wb_ref, b1_ref, w2_ref, b2_ref, out_ref):
    x = x_ref[...]
    inbox = i0_ref[...] + i1_ref[...]
    h = jnp.maximum(
        jnp.dot(x, wa_ref[...], preferred_element_type=jnp.float32)
        + jnp.dot(inbox, wb_ref[...], preferred_element_type=jnp.float32)
        + b1_ref[...], 0.0)
    out_ref[...] = x + jnp.dot(h, w2_ref[...], preferred_element_type=jnp.float32) + b2_ref[...]


BE = 2560


def kernel(nodes, edges, senders, receivers, edge_W1, edge_b1, edge_W2, edge_b2,
           node_W1, node_b1, node_W2, node_b2):
    x = nodes.reshape(N, D)
    e = edges.reshape(E, D)
    recv = receivers.astype(jnp.int32)
    send = senders.astype(jnp.int32)
    recv2 = recv.reshape(NS, EPT)
    send2 = send.reshape(NS, EPT)
    recv3 = recv.reshape(NW, SNCH, SCH)

    w1e = edge_W1[:D]
    w1r = edge_W1[D:2 * D]
    w1s = edge_W1[2 * D:]
    wn1a = node_W1[:D]
    wn1b = node_W1[D:]

    p_r, p_s = pl.pallas_call(
        _prep_body,
        out_shape=(jax.ShapeDtypeStruct((N, D), jnp.float32),
                   jax.ShapeDtypeStruct((N, D), jnp.float32)),
    )(x, w1r, w1s)

    sc_gather, sc_scatter = _build_sc_kernels()
    g_r, g_s = sc_gather(p_r.reshape(NS, RPT, D), p_s.reshape(NS, RPT, D),
                         recv2, send2)

    b1r = edge_b1.reshape(1, D)
    b2r = edge_b2.reshape(1, D)
    new_edges = pl.pallas_call(
        _edge_body,
        grid=(E // BE,),
        in_specs=[
            pl.BlockSpec((BE, D), lambda i: (i, 0)),
            pl.BlockSpec((BE, D), lambda i: (i, 0)),
            pl.BlockSpec((BE, D), lambda i: (i, 0)),
            pl.BlockSpec((D, D), lambda i: (0, 0)),
            pl.BlockSpec((1, D), lambda i: (0, 0)),
            pl.BlockSpec((D, D), lambda i: (0, 0)),
            pl.BlockSpec((1, D), lambda i: (0, 0)),
        ],
        out_specs=pl.BlockSpec((BE, D), lambda i: (i, 0)),
        out_shape=jax.ShapeDtypeStruct((E, D), jnp.float32),
    )(e, g_r, g_s, w1e, b1r, edge_W2, b2r)

    zrows = jnp.zeros((RPT, D), jnp.float32)
    partials = sc_scatter(new_edges, recv3, zrows).reshape(NC, N, D)

    bn1r = node_b1.reshape(1, D)
    bn2r = node_b2.reshape(1, D)
    BN = 2000
    new_nodes = pl.pallas_call(
        _node_body,
        grid=(N // BN,),
        in_specs=[
            pl.BlockSpec((BN, D), lambda i: (i, 0)),
            pl.BlockSpec((BN, D), lambda i: (i, 0)),
            pl.BlockSpec((BN, D), lambda i: (i, 0)),
            pl.BlockSpec((D, D), lambda i: (0, 0)),
            pl.BlockSpec((D, D), lambda i: (0, 0)),
            pl.BlockSpec((1, D), lambda i: (0, 0)),
            pl.BlockSpec((D, D), lambda i: (0, 0)),
            pl.BlockSpec((1, D), lambda i: (0, 0)),
        ],
        out_specs=pl.BlockSpec((BN, D), lambda i: (i, 0)),
        out_shape=jax.ShapeDtypeStruct((N, D), jnp.float32),
    )(x, partials[0], partials[1], wn1a, wn1b, bn1r, node_W2, bn2r)

    return (new_nodes.reshape(1, N, D), new_edges.reshape(1, E, D))
